# baseline (speedup 1.0000x reference)
import jax
import jax.numpy as jnp
from jax.experimental import pallas as pl
from jax.experimental.pallas import tpu as pltpu

ALPHA = 0.1
CC = 100
NBUF = 6


def _stream_kernel(x_hbm, o_hbm, xbuf, obuf, in_sem, out_sem):
    b, c, m, n = x_hbm.shape
    kc = c // CC
    nchunks = b * kc

    def in_copy(i, slot):
        ib = i // kc
        jc = i % kc
        return pltpu.make_async_copy(
            x_hbm.at[ib, pl.ds(jc * CC, CC)], xbuf.at[slot], in_sem.at[slot])

    def out_copy(i, slot):
        ib = i // kc
        jc = i % kc
        return pltpu.make_async_copy(
            obuf.at[slot], o_hbm.at[ib, pl.ds(jc * CC, CC)], out_sem.at[slot])

    for s in range(NBUF):
        in_copy(s, s).start()

    def body(i, carry):
        slot = jax.lax.rem(i, NBUF)
        in_copy(i, slot).wait()

        @pl.when(i >= NBUF)
        def _():
            out_copy(i - NBUF, slot).wait()

        blk = xbuf[slot]
        mx = jnp.max(blk, axis=(1, 2), keepdims=True)
        mn = jnp.min(blk, axis=(1, 2), keepdims=True)
        d = mx - mn
        factor = jnp.where((mx > mn) & jnp.isfinite(d),
                           jnp.float32(ALPHA), jnp.float32(1.0))
        obuf[slot] = blk * factor
        out_copy(i, slot).start()

        @pl.when(i + NBUF < nchunks)
        def _():
            in_copy(i + NBUF, slot).start()

        return carry

    jax.lax.fori_loop(0, nchunks, body, 0)

    for s in range(NBUF):
        i = nchunks - NBUF + s
        out_copy(i, i % NBUF).wait()


def kernel(cam):
    b, c, m, n = cam.shape
    return pl.pallas_call(
        _stream_kernel,
        in_specs=[pl.BlockSpec(memory_space=pltpu.MemorySpace.HBM)],
        out_specs=pl.BlockSpec(memory_space=pltpu.MemorySpace.HBM),
        out_shape=jax.ShapeDtypeStruct((b, c, m, n), cam.dtype),
        scratch_shapes=[
            pltpu.VMEM((NBUF, CC, m, n), jnp.float32),
            pltpu.VMEM((NBUF, CC, m, n), jnp.float32),
            pltpu.SemaphoreType.DMA((NBUF,)),
            pltpu.SemaphoreType.DMA((NBUF,)),
        ],
    )(cam)


# speedup vs baseline: 6.0267x; 6.0267x over previous
import jax
import jax.numpy as jnp
from jax.experimental import pallas as pl
from jax.experimental.pallas import tpu as pltpu

ALPHA = 0.1
B_TILE = 1


def _scale_kernel(x_ref, o_ref):
    blk = x_ref[...]
    mx = jnp.max(blk, axis=(1, 2), keepdims=True)
    mn = jnp.min(blk, axis=(1, 2), keepdims=True)
    d = mx - mn
    factor = jnp.where((mx > mn) & jnp.isfinite(d),
                       jnp.float32(ALPHA), jnp.float32(1.0))
    o_ref[...] = blk * factor


def kernel(cam):
    b, c, m, n = cam.shape
    t = jnp.transpose(cam, (0, 2, 3, 1))
    grid = (b // B_TILE,)
    out_t = pl.pallas_call(
        _scale_kernel,
        grid=grid,
        in_specs=[pl.BlockSpec((B_TILE, m, n, c), lambda i: (i, 0, 0, 0))],
        out_specs=pl.BlockSpec((B_TILE, m, n, c), lambda i: (i, 0, 0, 0)),
        out_shape=jax.ShapeDtypeStruct((b, m, n, c), cam.dtype),
        compiler_params=pltpu.CompilerParams(
            dimension_semantics=("parallel",)),
    )(t)
    return jnp.transpose(out_t, (0, 3, 1, 2))


# speedup vs baseline: 6.2825x; 1.0425x over previous
import jax
import jax.numpy as jnp
from jax.experimental import pallas as pl
from jax.experimental.pallas import tpu as pltpu

ALPHA = 0.1
B_TILE = 2


def _scale_kernel(x_ref, o_ref):
    blk = x_ref[...]
    mx = jnp.max(blk, axis=(1, 2), keepdims=True)
    mn = jnp.min(blk, axis=(1, 2), keepdims=True)
    d = mx - mn
    factor = jnp.where((mx > mn) & jnp.isfinite(d),
                       jnp.float32(ALPHA), jnp.float32(1.0))
    o_ref[...] = blk * factor


def kernel(cam):
    b, c, m, n = cam.shape
    t = jnp.transpose(cam, (0, 2, 3, 1))
    grid = (b // B_TILE,)
    out_t = pl.pallas_call(
        _scale_kernel,
        grid=grid,
        in_specs=[pl.BlockSpec((B_TILE, m, n, c), lambda i: (i, 0, 0, 0))],
        out_specs=pl.BlockSpec((B_TILE, m, n, c), lambda i: (i, 0, 0, 0)),
        out_shape=jax.ShapeDtypeStruct((b, m, n, c), cam.dtype),
        compiler_params=pltpu.CompilerParams(
            dimension_semantics=("parallel",)),
    )(t)
    return jnp.transpose(out_t, (0, 3, 1, 2))


# speedup vs baseline: 6.3462x; 1.0101x over previous
import jax
import jax.numpy as jnp
from jax.experimental import pallas as pl
from jax.experimental.pallas import tpu as pltpu

ALPHA = 0.1
B_TILE = 4


def _scale_kernel(x_ref, o_ref):
    blk = x_ref[...]
    mx = jnp.max(blk, axis=(1, 2), keepdims=True)
    mn = jnp.min(blk, axis=(1, 2), keepdims=True)
    d = mx - mn
    factor = jnp.where((mx > mn) & jnp.isfinite(d),
                       jnp.float32(ALPHA), jnp.float32(1.0))
    o_ref[...] = blk * factor


def kernel(cam):
    b, c, m, n = cam.shape
    t = jnp.transpose(cam, (0, 2, 3, 1))
    grid = (b // B_TILE,)
    out_t = pl.pallas_call(
        _scale_kernel,
        grid=grid,
        in_specs=[pl.BlockSpec((B_TILE, m, n, c), lambda i: (i, 0, 0, 0))],
        out_specs=pl.BlockSpec((B_TILE, m, n, c), lambda i: (i, 0, 0, 0)),
        out_shape=jax.ShapeDtypeStruct((b, m, n, c), cam.dtype),
        compiler_params=pltpu.CompilerParams(
            dimension_semantics=("parallel",)),
    )(t)
    return jnp.transpose(out_t, (0, 3, 1, 2))
